# baseline (speedup 1.0000x reference)
import functools
import math

import jax
import jax.numpy as jnp
from jax import lax
from jax.experimental import pallas as pl
from jax.experimental.pallas import tpu as pltpu
from jax.experimental.pallas import tpu_sc as plsc

EMBED_DIM = 64
SCALE = math.sqrt(EMBED_DIM)

B, T = 4096, 200
N = B * T
NUM_CORES = 2
NUM_SUBCORES = 16
NW = NUM_CORES * NUM_SUBCORES
ROWS_PER_W = N // NW
CHUNK = 128
NCHUNK = ROWS_PER_W // CHUNK


def _sc_embedding_lookup(x_flat, table):
    mesh = plsc.VectorSubcoreMesh(core_axis_name="c", subcore_axis_name="s")

    @functools.partial(
        pl.kernel,
        mesh=mesh,
        compiler_params=pltpu.CompilerParams(use_tc_tiling_on_sc=False),
        out_type=jax.ShapeDtypeStruct((N, EMBED_DIM), jnp.float32),
        scratch_types=[
            pltpu.VMEM((CHUNK,), jnp.int32),
            pltpu.VMEM((CHUNK, EMBED_DIM), jnp.float32),
            pltpu.SemaphoreType.DMA,
        ],
    )
    def k(idx_hbm, table_hbm, out_hbm, idx_v, rows_v, sem):
        wid = lax.axis_index("s") * NUM_CORES + lax.axis_index("c")
        base = wid * ROWS_PER_W

        def chunk_body(ci, _):
            start = base + ci * CHUNK
            pltpu.sync_copy(idx_hbm.at[pl.ds(start, CHUNK)], idx_v)
            pltpu.async_copy(table_hbm.at[idx_v], rows_v, sem).wait()

            def scale_body(i, _):
                for j in range(EMBED_DIM // 16):
                    sl = pl.ds(j * 16, 16)
                    rows_v[i, sl] = rows_v[i, sl] * SCALE
                return 0

            lax.fori_loop(0, CHUNK, scale_body, 0)
            pltpu.sync_copy(rows_v, out_hbm.at[pl.ds(start, CHUNK)])
            return 0

        lax.fori_loop(0, NCHUNK, chunk_body, 0)

    return k(x_flat, table)


def kernel(x, table):
    x_flat = x.reshape(N)
    out = _sc_embedding_lookup(x_flat, table)
    return out.reshape(B, T, EMBED_DIM)


# speedup vs baseline: 1.2736x; 1.2736x over previous
import functools
import math

import jax
import jax.numpy as jnp
from jax import lax
from jax.experimental import pallas as pl
from jax.experimental.pallas import tpu as pltpu
from jax.experimental.pallas import tpu_sc as plsc

EMBED_DIM = 64
SCALE = math.sqrt(EMBED_DIM)

B, T = 4096, 200
N = B * T
NUM_CORES = 2
NUM_SUBCORES = 16
NW = NUM_CORES * NUM_SUBCORES
ROWS_PER_W = N // NW
CHUNK = 128
NCHUNK = ROWS_PER_W // CHUNK
NBUF = 8
NGROUP = NCHUNK // NBUF


def _sc_embedding_lookup(x_flat, table):
    mesh = plsc.VectorSubcoreMesh(core_axis_name="c", subcore_axis_name="s")

    scratch = (
        [pltpu.VMEM((ROWS_PER_W,), jnp.int32)]
        + [pltpu.VMEM((CHUNK, EMBED_DIM), jnp.float32)] * NBUF
        + [pltpu.SemaphoreType.DMA] * NBUF
    )

    @functools.partial(
        pl.kernel,
        mesh=mesh,
        compiler_params=pltpu.CompilerParams(use_tc_tiling_on_sc=False),
        out_type=jax.ShapeDtypeStruct((N, EMBED_DIM), jnp.float32),
        scratch_types=scratch,
    )
    def k(idx_hbm, table_hbm, out_hbm, idx_v, *bufs_and_sems):
        rows = bufs_and_sems[:NBUF]
        sems = bufs_and_sems[NBUF:]
        wid = lax.axis_index("s") * NUM_CORES + lax.axis_index("c")
        base = wid * ROWS_PER_W

        pltpu.sync_copy(idx_hbm.at[pl.ds(base, ROWS_PER_W)], idx_v)

        def fire(ci, b):
            src = table_hbm.at[idx_v.at[pl.ds(ci * CHUNK, CHUNK)]]
            pltpu.async_copy(src, rows[b], sems[b])

        def drain(ci, b):
            src = table_hbm.at[idx_v.at[pl.ds(ci * CHUNK, CHUNK)]]
            pltpu.make_async_copy(src, rows[b], sems[b]).wait()

        def scale_and_store(ci, b):
            @plsc.parallel_loop(0, CHUNK, step=1, unroll=4)
            def _(r):
                for j in range(EMBED_DIM // 16):
                    sl = pl.ds(j * 16, 16)
                    rows[b][r, sl] = rows[b][r, sl] * SCALE

            pltpu.sync_copy(rows[b], out_hbm.at[pl.ds(base + ci * CHUNK, CHUNK)])

        for b in range(NBUF):
            fire(b, b)

        def group_body(g, _):
            for b in range(NBUF):
                ci = g * NBUF + b
                drain(ci, b)
                scale_and_store(ci, b)
                fire(ci + NBUF, b)
            return 0

        lax.fori_loop(0, NGROUP - 1, group_body, 0)

        for b in range(NBUF):
            ci = (NGROUP - 1) * NBUF + b
            drain(ci, b)
            scale_and_store(ci, b)

    return k(x_flat, table)


def kernel(x, table):
    x_flat = x.reshape(N)
    out = _sc_embedding_lookup(x_flat, table)
    return out.reshape(B, T, EMBED_DIM)
